# baseline (speedup 1.0000x reference)
import functools

import jax
import jax.numpy as jnp
from jax import lax
from jax.experimental import pallas as pl
from jax.experimental.pallas import tpu as pltpu
from jax.experimental.pallas import tpu_sc as plsc

B = 1024
OTHER_LEN = 50
LANES_LEN = 200
BDRY_LEN = 200
EMB_DIM = 64
TOTAL_LEN = 1 + OTHER_LEN + LANES_LEN + BDRY_LEN
ROW = TOTAL_LEN * EMB_DIM
AOAT_PAD = 64

_NC = 2
_NS = 16
_NW = _NC * _NS
_SCENES_PER_W = B // _NW


def _sc_body(aoat_hbm, w_hbm, out_hbm, aoat_v, w_v, tmpl, sem):
    wid = lax.axis_index("s") * _NC + lax.axis_index("c")
    base = wid * _SCENES_PER_W

    pltpu.sync_copy(w_hbm, w_v)
    pltpu.sync_copy(
        aoat_hbm.at[pl.ds(base * AOAT_PAD, _SCENES_PER_W * AOAT_PAD)], aoat_v)

    nch = EMB_DIM // 16
    w0c = [w_v[pl.ds(0 * EMB_DIM + 16 * c, 16)] for c in range(nch)]
    w1c = [w_v[pl.ds(1 * EMB_DIM + 16 * c, 16)] for c in range(nch)]
    w2c = [w_v[pl.ds(2 * EMB_DIM + 16 * c, 16)] for c in range(nch)]
    w5c = [w_v[pl.ds(5 * EMB_DIM + 16 * c, 16)] for c in range(nch)]
    w11c = [w_v[pl.ds(11 * EMB_DIM + 16 * c, 16)] for c in range(nch)]

    for c in range(nch):
        tmpl[0, pl.ds(16 * c, 16)] = w0c[c]

    def _fill(lo, hi, chunks):
        def body(r, carry):
            for c in range(nch):
                tmpl[r, pl.ds(16 * c, 16)] = chunks[c]
            return carry
        lax.fori_loop(lo, hi, body, 0)

    _fill(1 + OTHER_LEN, 1 + OTHER_LEN + LANES_LEN, w5c)
    _fill(1 + OTHER_LEN + LANES_LEN, TOTAL_LEN, w11c)

    def scene_body(s, carry):
        for j in range(OTHER_LEN):
            sel = plsc.load_gather(
                aoat_v, [jnp.full((16,), s * AOAT_PAD + j, jnp.int32)])
            m = sel == 1
            for c in range(nch):
                tmpl[1 + j, pl.ds(16 * c, 16)] = jnp.where(
                    m, w2c[c], w1c[c])
        pltpu.sync_copy(tmpl, out_hbm.at[base + s])
        return carry

    lax.fori_loop(0, _SCENES_PER_W, scene_body, 0)


@functools.partial(jax.jit, static_argnames=())
def _sc_call(aoat_flat, w_flat):
    mesh = plsc.VectorSubcoreMesh(core_axis_name="c", subcore_axis_name="s")
    f = functools.partial(
        pl.kernel,
        out_type=jax.ShapeDtypeStruct((B, TOTAL_LEN, EMB_DIM), jnp.float32),
        mesh=mesh,
        compiler_params=pltpu.CompilerParams(
            needs_layout_passes=False, use_tc_tiling_on_sc=True),
        scratch_types=[
            pltpu.VMEM((_SCENES_PER_W * AOAT_PAD,), jnp.int32),
            pltpu.VMEM((12 * EMB_DIM,), jnp.float32),
            pltpu.VMEM((TOTAL_LEN, EMB_DIM), jnp.float32),
            pltpu.SemaphoreType.DMA,
        ],
    )(_sc_body)
    return f(aoat_flat, w_flat)


def kernel(type, all_other_agents_types, lanes_mid, lanes, embedding_weight):
    del type, lanes_mid, lanes
    aoat = all_other_agents_types.astype(jnp.int32)
    aoat_flat = jnp.pad(
        aoat, ((0, 0), (0, AOAT_PAD - OTHER_LEN))).reshape(-1)
    w_flat = embedding_weight.reshape(-1)
    return _sc_call(aoat_flat, w_flat)


# speedup vs baseline: 3.5400x; 3.5400x over previous
import functools

import jax
import jax.numpy as jnp
from jax import lax
from jax.experimental import pallas as pl
from jax.experimental.pallas import tpu as pltpu
from jax.experimental.pallas import tpu_sc as plsc

B = 1024
OTHER_LEN = 50
EMB_DIM = 64
TOTAL_LEN = 451
LANES_START = 1 + OTHER_LEN
BDRY_START = LANES_START + 200

_NC = 2
_NS = 16
_NW = _NC * _NS
_EG = 8
_NCH = B // 16


def _splat(w_v, t, e):
    return plsc.load_gather(
        w_v, [jnp.full((16,), t * EMB_DIM + e, jnp.int32)])


def _build_const(tmpl, w_v, t):
    for eg in range(0, EMB_DIM, _EG):
        sp = [_splat(w_v, t, eg + i) for i in range(_EG)]

        def body(c, carry):
            for i in range(_EG):
                tmpl[eg + i, pl.ds(16 * c, 16)] = sp[i]
            return carry
        lax.fori_loop(0, _NCH, body, 0)


def _build_agent(tmpl, w_v, arow):
    for eg in range(0, EMB_DIM, _EG):
        sp1 = [_splat(w_v, 1, eg + i) for i in range(_EG)]
        sp2 = [_splat(w_v, 2, eg + i) for i in range(_EG)]

        def body(c, carry):
            a = arow[pl.ds(16 * c, 16)]
            m = a == 1
            for i in range(_EG):
                tmpl[eg + i, pl.ds(16 * c, 16)] = jnp.where(m, sp2[i], sp1[i])
            return carry
        lax.fori_loop(0, _NCH, body, 0)


def _sc_body(aoat_hbm, w_hbm, out_hbm, arow_v, w_v, tmpl, sem, asem):
    wid = lax.axis_index("s") * _NC + lax.axis_index("c")

    pltpu.sync_copy(w_hbm, w_v)

    def _fire_range(lo, hi):
        first = wid + _NW * ((lo - wid + _NW - 1) // _NW)
        n = (hi - first + _NW - 1) // _NW

        def fire(k, carry):
            pltpu.make_async_copy(tmpl, out_hbm.at[first + _NW * k], sem).start()
            return carry
        lax.fori_loop(0, n, fire, 0)

        def drain(k, carry):
            pltpu.make_async_copy(tmpl, out_hbm.at[first], sem).wait()
            return carry
        lax.fori_loop(0, n, drain, 0)
        return first, n

    _build_const(tmpl, w_v, 5)
    _fire_range(LANES_START, BDRY_START)
    _build_const(tmpl, w_v, 11)
    _fire_range(BDRY_START, TOTAL_LEN)

    @pl.when(wid == 0)
    def _():
        _build_const(tmpl, w_v, 0)
        pltpu.sync_copy(tmpl, out_hbm.at[0])

    first = jnp.where(wid == 0, _NW, wid)
    n_agent = (OTHER_LEN - first + _NW) // _NW

    def agent(k, carry):
        p = first + _NW * k
        pltpu.sync_copy(aoat_hbm.at[p - 1], arow_v)
        _build_agent(tmpl, w_v, arow_v)
        pltpu.sync_copy(tmpl, out_hbm.at[p])
        return carry
    lax.fori_loop(0, n_agent, agent, 0)


@jax.jit
def _sc_call(aoat_t, w_flat):
    mesh = plsc.VectorSubcoreMesh(core_axis_name="c", subcore_axis_name="s")
    f = functools.partial(
        pl.kernel,
        out_type=jax.ShapeDtypeStruct((TOTAL_LEN, EMB_DIM, B), jnp.float32),
        mesh=mesh,
        compiler_params=pltpu.CompilerParams(
            needs_layout_passes=False, use_tc_tiling_on_sc=True),
        scratch_types=[
            pltpu.VMEM((B,), jnp.int32),
            pltpu.VMEM((12 * EMB_DIM,), jnp.float32),
            pltpu.VMEM((EMB_DIM, B), jnp.float32),
            pltpu.SemaphoreType.DMA,
            pltpu.SemaphoreType.DMA,
        ],
    )(_sc_body)
    return f(aoat_t, w_flat)


def kernel(type, all_other_agents_types, lanes_mid, lanes, embedding_weight):
    del type, lanes_mid, lanes
    aoat_t = all_other_agents_types.astype(jnp.int32).T
    w_flat = embedding_weight.reshape(-1)
    out_t = _sc_call(aoat_t, w_flat)
    return jnp.transpose(out_t, (2, 0, 1))


# speedup vs baseline: 3.5409x; 1.0002x over previous
import functools

import jax
import jax.numpy as jnp
from jax import lax
from jax.experimental import pallas as pl
from jax.experimental.pallas import tpu as pltpu
from jax.experimental.pallas import tpu_sc as plsc

B = 1024
OTHER_LEN = 50
EMB_DIM = 64
TOTAL_LEN = 451
LANES_START = 1 + OTHER_LEN
BDRY_START = LANES_START + 200

_NC = 2
_NS = 16
_NW = _NC * _NS
_EG = 8
_NCH = B // 16


def _splat(w_v, t, e):
    return plsc.load_gather(
        w_v, [jnp.full((16,), 16 + t * EMB_DIM + e, jnp.int32)])


def _build_const(tmpl, w_v, t):
    for eg in range(0, EMB_DIM, _EG):
        sp = [_splat(w_v, t, eg + i) for i in range(_EG)]

        def body(c, carry):
            for i in range(_EG):
                tmpl[eg + i, pl.ds(16 * c, 16)] = sp[i]
            return carry
        lax.fori_loop(0, _NCH, body, 0)


def _build_agent(tmpl, w_v, arow):
    for eg in range(0, EMB_DIM, _EG):
        sp1 = [_splat(w_v, 1, eg + i) for i in range(_EG)]
        sp2 = [_splat(w_v, 2, eg + i) for i in range(_EG)]

        def body(c, carry):
            a = arow[pl.ds(16 * c, 16)]
            m = a == 1
            for i in range(_EG):
                tmpl[eg + i, pl.ds(16 * c, 16)] = jnp.where(m, sp2[i], sp1[i])
            return carry
        lax.fori_loop(0, _NCH, body, 0)


def _sc_body(aoat_hbm, w_hbm, out_hbm, arow_v, w_v, tmpl, sem, asem):
    wid = lax.axis_index("s") * _NC + lax.axis_index("c")

    pltpu.sync_copy(w_hbm, w_v.at[pl.ds(16, 12 * EMB_DIM)])

    def _fire_range(lo, hi):
        first = wid + _NW * ((lo - wid + _NW - 1) // _NW)
        n = (hi - first + _NW - 1) // _NW

        def fire(k, carry):
            pltpu.make_async_copy(tmpl, out_hbm.at[first + _NW * k], sem).start()
            return carry
        lax.fori_loop(0, n, fire, 0)

        def drain(k, carry):
            pltpu.make_async_copy(tmpl, out_hbm.at[first], sem).wait()
            return carry
        lax.fori_loop(0, n, drain, 0)
        return first, n

    _build_const(tmpl, w_v, 5)
    _fire_range(LANES_START, BDRY_START)
    _build_const(tmpl, w_v, 11)
    _fire_range(BDRY_START, TOTAL_LEN)

    @pl.when(wid == 0)
    def _():
        _build_const(tmpl, w_v, 0)
        pltpu.sync_copy(tmpl, out_hbm.at[0])

    first = jnp.where(wid == 0, _NW, wid)
    n_agent = (OTHER_LEN - first + _NW) // _NW

    def agent(k, carry):
        p = first + _NW * k
        pltpu.sync_copy(aoat_hbm.at[p - 1], arow_v)
        _build_agent(tmpl, w_v, arow_v)
        pltpu.sync_copy(tmpl, out_hbm.at[p])
        return carry
    lax.fori_loop(0, n_agent, agent, 0)


@jax.jit
def _sc_call(aoat_t, w_flat):
    mesh = plsc.VectorSubcoreMesh(core_axis_name="c", subcore_axis_name="s")
    f = functools.partial(
        pl.kernel,
        out_type=jax.ShapeDtypeStruct((TOTAL_LEN, EMB_DIM, B), jnp.float32),
        mesh=mesh,
        compiler_params=pltpu.CompilerParams(
            needs_layout_passes=False, use_tc_tiling_on_sc=True),
        scratch_types=[
            pltpu.VMEM((B,), jnp.int32),
            pltpu.VMEM((16 + 12 * EMB_DIM,), jnp.float32),
            pltpu.VMEM((EMB_DIM, B), jnp.float32),
            pltpu.SemaphoreType.DMA,
            pltpu.SemaphoreType.DMA,
        ],
    )(_sc_body)
    return f(aoat_t, w_flat)


def kernel(type, all_other_agents_types, lanes_mid, lanes, embedding_weight):
    del type, lanes_mid, lanes
    aoat_t = all_other_agents_types.astype(jnp.int32).T
    w_flat = embedding_weight.reshape(-1)
    out_t = _sc_call(aoat_t, w_flat)
    return jnp.transpose(out_t, (2, 0, 1))


# speedup vs baseline: 4.0432x; 1.1419x over previous
import functools

import jax
import jax.numpy as jnp
from jax import lax
from jax.experimental import pallas as pl
from jax.experimental.pallas import tpu as pltpu
from jax.experimental.pallas import tpu_sc as plsc

B = 1024
HALF = B // 2
OTHER_LEN = 50
EMB_DIM = 64
TOTAL_LEN = 451
LANES_START = 1 + OTHER_LEN
BDRY_START = LANES_START + 200

_NC = 2
_NS = 16
_NW = _NC * _NS
_EG = 8
_WOFF = 16


def _splat(w_v, t, e):
    return plsc.load_gather(
        w_v, [jnp.full((16,), _WOFF + t * EMB_DIM + e, jnp.int32)])


def _build_const(panel, w_v, t, width):
    nch = width // 16
    for eg in range(0, EMB_DIM, _EG):
        sp = [_splat(w_v, t, eg + i) for i in range(_EG)]

        def body(c, carry):
            for i in range(_EG):
                panel[eg + i, pl.ds(16 * c, 16)] = sp[i]
            return carry
        lax.fori_loop(0, nch, body, 0)


def _build_agent(tmpl, w_v, arow):
    nch = B // 16
    for eg in range(0, EMB_DIM, _EG):
        sp1 = [_splat(w_v, 1, eg + i) for i in range(_EG)]
        sp2 = [_splat(w_v, 2, eg + i) for i in range(_EG)]

        def body(c, carry):
            a = arow[pl.ds(16 * c, 16)]
            m = a == 1
            for i in range(_EG):
                tmpl[eg + i, pl.ds(16 * c, 16)] = jnp.where(m, sp2[i], sp1[i])
            return carry
        lax.fori_loop(0, nch, body, 0)


def _sc_body(aoat_hbm, w_hbm, out_hbm, arow_v, w_v, cpan, atmpl, sem, asem):
    wid = lax.axis_index("s") * _NC + lax.axis_index("c")

    pltpu.sync_copy(w_hbm, w_v.at[pl.ds(_WOFF, 12 * EMB_DIM)])

    def _range(lo, hi):
        first = wid + _NW * ((lo - wid + _NW - 1) // _NW)
        n = (hi - first + _NW - 1) // _NW
        return first, n

    def _fire_halves(first, n):
        def fire(k, carry):
            p = first + _NW * k
            pltpu.make_async_copy(
                cpan, out_hbm.at[p, :, pl.ds(0, HALF)], sem).start()
            pltpu.make_async_copy(
                cpan, out_hbm.at[p, :, pl.ds(HALF, HALF)], sem).start()
            return carry
        lax.fori_loop(0, n, fire, 0)

    def _drain_halves(first, n):
        def drain(k, carry):
            pltpu.make_async_copy(
                cpan, out_hbm.at[first, :, pl.ds(0, HALF)], sem).wait()
            pltpu.make_async_copy(
                cpan, out_hbm.at[first, :, pl.ds(HALF, HALF)], sem).wait()
            return carry
        lax.fori_loop(0, n, drain, 0)

    first5, n5 = _range(LANES_START, BDRY_START)
    first11, n11 = _range(BDRY_START, TOTAL_LEN)

    first_a = jnp.where(wid == 0, _NW, wid)
    n_agent = (OTHER_LEN - first_a + _NW) // _NW

    _build_const(cpan, w_v, 5, HALF)
    _fire_halves(first5, n5)

    pltpu.sync_copy(aoat_hbm.at[first_a - 1], arow_v)
    _build_agent(atmpl, w_v, arow_v)
    pltpu.make_async_copy(atmpl, out_hbm.at[first_a], asem).start()

    _drain_halves(first5, n5)
    _build_const(cpan, w_v, 11, HALF)
    _fire_halves(first11, n11)

    @pl.when(n_agent == 2)
    def _():
        p2 = first_a + _NW
        pltpu.make_async_copy(atmpl, out_hbm.at[first_a], asem).wait()
        pltpu.sync_copy(aoat_hbm.at[p2 - 1], arow_v)
        _build_agent(atmpl, w_v, arow_v)
        pltpu.make_async_copy(atmpl, out_hbm.at[p2], asem).start()

    _drain_halves(first11, n11)

    @pl.when(wid == 0)
    def _():
        _build_const(cpan, w_v, 0, HALF)
        pltpu.make_async_copy(
            cpan, out_hbm.at[0, :, pl.ds(0, HALF)], sem).start()
        pltpu.make_async_copy(
            cpan, out_hbm.at[0, :, pl.ds(HALF, HALF)], sem).start()
        _drain_halves(0, 1)

    pltpu.make_async_copy(atmpl, out_hbm.at[first_a], asem).wait()


@jax.jit
def _sc_call(aoat_t, w_flat):
    mesh = plsc.VectorSubcoreMesh(core_axis_name="c", subcore_axis_name="s")
    f = functools.partial(
        pl.kernel,
        out_type=jax.ShapeDtypeStruct((TOTAL_LEN, EMB_DIM, B), jnp.float32),
        mesh=mesh,
        compiler_params=pltpu.CompilerParams(
            needs_layout_passes=False, use_tc_tiling_on_sc=True),
        scratch_types=[
            pltpu.VMEM((B,), jnp.int32),
            pltpu.VMEM((_WOFF + 12 * EMB_DIM,), jnp.float32),
            pltpu.VMEM((EMB_DIM, HALF), jnp.float32),
            pltpu.VMEM((EMB_DIM, B), jnp.float32),
            pltpu.SemaphoreType.DMA,
            pltpu.SemaphoreType.DMA,
        ],
    )(_sc_body)
    return f(aoat_t, w_flat)


def kernel(type, all_other_agents_types, lanes_mid, lanes, embedding_weight):
    del type, lanes_mid, lanes
    aoat_t = all_other_agents_types.astype(jnp.int32).T
    w_flat = embedding_weight.reshape(-1)
    out_t = _sc_call(aoat_t, w_flat)
    return jnp.transpose(out_t, (2, 0, 1))


# speedup vs baseline: 4.1412x; 1.0242x over previous
import functools

import jax
import jax.numpy as jnp
from jax import lax
from jax.experimental import pallas as pl
from jax.experimental.pallas import tpu as pltpu
from jax.experimental.pallas import tpu_sc as plsc

B = 1024
HALF = B // 2
OTHER_LEN = 50
EMB_DIM = 64
TOTAL_LEN = 451
LANES_START = 1 + OTHER_LEN
BDRY_START = LANES_START + 200

_NC = 2
_NS = 16
_NW = _NC * _NS
_WOFF = 16


def _splat(w_v, t, e):
    return plsc.load_gather(
        w_v, [jnp.full((16,), _WOFF + t * EMB_DIM + e, jnp.int32)])


def _build_const(panel, w_v, t, width, eg_sz=8):
    nch = width // 16
    for eg in range(0, EMB_DIM, eg_sz):
        sp = [_splat(w_v, t, eg + i) for i in range(eg_sz)]

        def body(c, carry):
            for i in range(eg_sz):
                panel[eg + i, pl.ds(16 * c, 16)] = sp[i]
            return carry
        lax.fori_loop(0, nch, body, 0)


def _build_agent(tmpl, w_v, arow, eg_sz=16):
    nch = B // 16
    for eg in range(0, EMB_DIM, eg_sz):
        sp1 = [_splat(w_v, 1, eg + i) for i in range(eg_sz)]
        sp2 = [_splat(w_v, 2, eg + i) for i in range(eg_sz)]

        def body(c, carry):
            a = arow[pl.ds(16 * c, 16)]
            m = a == 1
            for i in range(eg_sz):
                tmpl[eg + i, pl.ds(16 * c, 16)] = jnp.where(m, sp2[i], sp1[i])
            return carry
        lax.fori_loop(0, nch, body, 0)


def _sc_body(aoat_hbm, w_hbm, out_hbm, arow_v, w_v, cpan, atmpl, sem, asem):
    wid = lax.axis_index("s") * _NC + lax.axis_index("c")

    pltpu.sync_copy(w_hbm, w_v.at[pl.ds(_WOFF, 12 * EMB_DIM)])

    def _range(lo, hi):
        first = wid + _NW * ((lo - wid + _NW - 1) // _NW)
        n = (hi - first + _NW - 1) // _NW
        return first, n

    def _fire_halves(first, n):
        def fire(k, carry):
            p = first + _NW * k
            pltpu.make_async_copy(
                cpan, out_hbm.at[p, :, pl.ds(0, HALF)], sem).start()
            pltpu.make_async_copy(
                cpan, out_hbm.at[p, :, pl.ds(HALF, HALF)], sem).start()
            return carry
        lax.fori_loop(0, n, fire, 0)

    def _drain_halves(first, n):
        def drain(k, carry):
            pltpu.make_async_copy(
                cpan, out_hbm.at[first, :, pl.ds(0, HALF)], sem).wait()
            pltpu.make_async_copy(
                cpan, out_hbm.at[first, :, pl.ds(HALF, HALF)], sem).wait()
            return carry
        lax.fori_loop(0, n, drain, 0)

    first5, n5 = _range(LANES_START, BDRY_START)
    first11, n11 = _range(BDRY_START, TOTAL_LEN)

    q = (wid + _NW - 2) % _NW
    first_a = jnp.where(q == 0, _NW, q)
    n_agent = jnp.where((q >= 1) & (q <= OTHER_LEN - _NW), 2, 1)

    _build_const(cpan, w_v, 5, HALF)
    _fire_halves(first5, n5)

    pltpu.sync_copy(aoat_hbm.at[first_a - 1], arow_v)
    _build_agent(atmpl, w_v, arow_v)
    pltpu.make_async_copy(atmpl, out_hbm.at[first_a], asem).start()

    _drain_halves(first5, n5)
    _build_const(cpan, w_v, 11, HALF)
    _fire_halves(first11, n11)

    @pl.when(n_agent == 2)
    def _():
        p2 = first_a + _NW
        pltpu.make_async_copy(atmpl, out_hbm.at[first_a], asem).wait()
        pltpu.sync_copy(aoat_hbm.at[p2 - 1], arow_v)
        _build_agent(atmpl, w_v, arow_v)
        pltpu.make_async_copy(atmpl, out_hbm.at[p2], asem).start()

    @pl.when(wid == 0)
    def _():
        pltpu.make_async_copy(atmpl, out_hbm.at[first_a], asem).wait()
        _build_const(atmpl, w_v, 0, B)
        pltpu.make_async_copy(atmpl, out_hbm.at[0], asem).start()

    _drain_halves(first11, n11)
    pltpu.make_async_copy(atmpl, out_hbm.at[first_a], asem).wait()


@jax.jit
def _sc_call(aoat_t, w_flat):
    mesh = plsc.VectorSubcoreMesh(core_axis_name="c", subcore_axis_name="s")
    f = functools.partial(
        pl.kernel,
        out_type=jax.ShapeDtypeStruct((TOTAL_LEN, EMB_DIM, B), jnp.float32),
        mesh=mesh,
        compiler_params=pltpu.CompilerParams(
            needs_layout_passes=False, use_tc_tiling_on_sc=True),
        scratch_types=[
            pltpu.VMEM((B,), jnp.int32),
            pltpu.VMEM((_WOFF + 12 * EMB_DIM,), jnp.float32),
            pltpu.VMEM((EMB_DIM, HALF), jnp.float32),
            pltpu.VMEM((EMB_DIM, B), jnp.float32),
            pltpu.SemaphoreType.DMA,
            pltpu.SemaphoreType.DMA,
        ],
    )(_sc_body)
    return f(aoat_t, w_flat)


def kernel(type, all_other_agents_types, lanes_mid, lanes, embedding_weight):
    del type, lanes_mid, lanes
    aoat_t = all_other_agents_types.astype(jnp.int32).T
    w_flat = embedding_weight.reshape(-1)
    out_t = _sc_call(aoat_t, w_flat)
    return jnp.transpose(out_t, (2, 0, 1))


# speedup vs baseline: 4.1740x; 1.0079x over previous
import functools

import jax
import jax.numpy as jnp
from jax import lax
from jax.experimental import pallas as pl
from jax.experimental.pallas import tpu as pltpu
from jax.experimental.pallas import tpu_sc as plsc

B = 1024
HALF = B // 2
OTHER_LEN = 50
EMB_DIM = 64
TOTAL_LEN = 451
LANES_START = 1 + OTHER_LEN
BDRY_START = LANES_START + 200

_NC = 2
_NS = 16
_NW = _NC * _NS
_WOFF = 16


def _splat(w_v, t, e):
    return plsc.load_gather(
        w_v, [jnp.full((16,), _WOFF + t * EMB_DIM + e, jnp.int32)])


def _build_const(panel, w_v, t, width, eg_sz=8):
    nch = width // 16
    for eg in range(0, EMB_DIM, eg_sz):
        sp = [_splat(w_v, t, eg + i) for i in range(eg_sz)]

        def body(c, carry):
            for i in range(eg_sz):
                panel[eg + i, pl.ds(16 * c, 16)] = sp[i]
            return carry
        lax.fori_loop(0, nch, body, 0)


def _build_agent(tmpl, w_v, arow, eg_sz=16):
    nch = B // 16
    for eg in range(0, EMB_DIM, eg_sz):
        sp1 = [_splat(w_v, 1, eg + i) for i in range(eg_sz)]
        sp2 = [_splat(w_v, 2, eg + i) for i in range(eg_sz)]

        def body(c, carry):
            a = arow[pl.ds(16 * c, 16)]
            m = a == 1
            for i in range(eg_sz):
                tmpl[eg + i, pl.ds(16 * c, 16)] = jnp.where(m, sp2[i], sp1[i])
            return carry
        lax.fori_loop(0, nch, body, 0)


def _sc_body(aoat_hbm, w_hbm, out_hbm, arow_v, w_v, cpan5, cpan11, atmpl,
             sem, asem):
    wid = lax.axis_index("s") * _NC + lax.axis_index("c")

    pltpu.sync_copy(w_hbm, w_v.at[pl.ds(_WOFF, 12 * EMB_DIM)])

    def _range(lo, hi):
        first = wid + _NW * ((lo - wid + _NW - 1) // _NW)
        n = (hi - first + _NW - 1) // _NW
        return first, n

    def _fire_panels(panel, width, first, n, start):
        nrep = B // width

        def fire(k, carry):
            p = first + _NW * k
            for h in range(nrep):
                cp = pltpu.make_async_copy(
                    panel, out_hbm.at[p, :, pl.ds(h * width, width)], sem)
                cp.start() if start else cp.wait()
            return carry
        lax.fori_loop(0, n, fire, 0)

    first5, n5 = _range(LANES_START, BDRY_START)
    first11, n11 = _range(BDRY_START, TOTAL_LEN)

    q = (wid + _NW - 2) % _NW
    first_a = jnp.where(q == 0, _NW, q)
    n_agent = jnp.where((q >= 1) & (q <= OTHER_LEN - _NW), 2, 1)

    _build_const(cpan5, w_v, 5, HALF)
    _fire_panels(cpan5, HALF, first5, n5, True)

    pltpu.sync_copy(aoat_hbm.at[first_a - 1], arow_v)
    _build_agent(atmpl, w_v, arow_v)
    pltpu.make_async_copy(atmpl, out_hbm.at[first_a], asem).start()

    _build_const(cpan11, w_v, 11, B // 4)
    _fire_panels(cpan11, B // 4, first11, n11, True)

    @pl.when(n_agent == 2)
    def _():
        p2 = first_a + _NW
        pltpu.make_async_copy(atmpl, out_hbm.at[first_a], asem).wait()
        pltpu.sync_copy(aoat_hbm.at[p2 - 1], arow_v)
        _build_agent(atmpl, w_v, arow_v)
        pltpu.make_async_copy(atmpl, out_hbm.at[p2], asem).start()

    @pl.when(wid == 0)
    def _():
        pltpu.make_async_copy(atmpl, out_hbm.at[first_a], asem).wait()
        _build_const(atmpl, w_v, 0, B)
        pltpu.make_async_copy(atmpl, out_hbm.at[0], asem).start()

    _fire_panels(cpan5, HALF, first5, n5, False)
    _fire_panels(cpan11, B // 4, first11, n11, False)
    pltpu.make_async_copy(atmpl, out_hbm.at[first_a], asem).wait()


@jax.jit
def _sc_call(aoat_t, w_flat):
    mesh = plsc.VectorSubcoreMesh(core_axis_name="c", subcore_axis_name="s")
    f = functools.partial(
        pl.kernel,
        out_type=jax.ShapeDtypeStruct((TOTAL_LEN, EMB_DIM, B), jnp.float32),
        mesh=mesh,
        compiler_params=pltpu.CompilerParams(
            needs_layout_passes=False, use_tc_tiling_on_sc=True),
        scratch_types=[
            pltpu.VMEM((B,), jnp.int32),
            pltpu.VMEM((_WOFF + 12 * EMB_DIM,), jnp.float32),
            pltpu.VMEM((EMB_DIM, HALF), jnp.float32),
            pltpu.VMEM((EMB_DIM, B // 4), jnp.float32),
            pltpu.VMEM((EMB_DIM, B), jnp.float32),
            pltpu.SemaphoreType.DMA,
            pltpu.SemaphoreType.DMA,
        ],
    )(_sc_body)
    return f(aoat_t, w_flat)


def kernel(type, all_other_agents_types, lanes_mid, lanes, embedding_weight):
    del type, lanes_mid, lanes
    aoat_t = all_other_agents_types.astype(jnp.int32).T
    w_flat = embedding_weight.reshape(-1)
    out_t = _sc_call(aoat_t, w_flat)
    return jnp.transpose(out_t, (2, 0, 1))
